# baseline (speedup 1.0000x reference)
import functools

import jax
import jax.numpy as jnp
from jax import lax
from jax.experimental import pallas as pl
from jax.experimental.pallas import tpu as pltpu
from jax.experimental.pallas import tpu_sc as plsc

NC = 2
NS = 16
NW = NC * NS
L = 16
CH = 64


@functools.cache
def _build(B, H, W, C, N):
    HW = H * W
    full_per_b = N // CH
    tail = N - full_per_b * CH
    n_main = B * full_per_b
    assert n_main % NW == 0, (n_main, NW)
    per_w = n_main // NW
    assert tail % 8 == 0 and tail < CH
    sx = float(W) / float(W - 1)
    sy = float(H) / float(H - 1)

    mesh = plsc.VectorSubcoreMesh(
        core_axis_name="c", subcore_axis_name="s", num_cores=NC, num_subcores=NS
    )

    @functools.partial(
        pl.kernel,
        out_type=jax.ShapeDtypeStruct((B, N, C), jnp.float32),
        mesh=mesh,
        scratch_types=[
            pltpu.VMEM((CH,), jnp.float32),
            pltpu.VMEM((CH,), jnp.float32),
            pltpu.VMEM((4, CH), jnp.int32),
            pltpu.VMEM((4, CH), jnp.float32),
            pltpu.VMEM((4 * CH, C), jnp.float32),
            pltpu.VMEM((CH, C), jnp.float32),
            pltpu.SemaphoreType.DMA,
        ],
        compiler_params=pltpu.CompilerParams(use_tc_tiling_on_sc=False),
    )
    def interp(x_hbm, px_hbm, py_hbm, out_hbm, pxv, pyv, idxv, wv, rowsv, outv, sem):
        wid = lax.axis_index("s") * NC + lax.axis_index("c")

        def do_chunk(b, lc, size):
            pbase = b * N + lc * CH
            pltpu.sync_copy(px_hbm.at[pl.ds(pbase, size)], pxv.at[pl.ds(0, size)])
            pltpu.sync_copy(py_hbm.at[pl.ds(pbase, size)], pyv.at[pl.ds(0, size)])

            for g in range(size // L):
                sl = pl.ds(g * L, L)
                px = pxv[sl]
                py = pyv[sl]
                ix = px * sx - 0.5
                iy = py * sy - 0.5
                fx0 = (ix + 1.0).astype(jnp.int32).astype(jnp.float32) - 1.0
                fy0 = (iy + 1.0).astype(jnp.int32).astype(jnp.float32) - 1.0
                wx1 = ix - fx0
                wx0 = 1.0 - wx1
                wy1 = iy - fy0
                wy0 = 1.0 - wy1
                fx1 = fx0 + 1.0
                fy1 = fy0 + 1.0
                mx0 = (fx0 >= 0.0) & (fx0 <= W - 1.0)
                mx1 = (fx1 >= 0.0) & (fx1 <= W - 1.0)
                my0 = (fy0 >= 0.0) & (fy0 <= H - 1.0)
                my1 = (fy1 >= 0.0) & (fy1 <= H - 1.0)
                cx0 = jnp.clip(fx0.astype(jnp.int32), 0, W - 1)
                cx1 = jnp.clip(fx1.astype(jnp.int32), 0, W - 1)
                cy0 = jnp.clip(fy0.astype(jnp.int32), 0, H - 1)
                cy1 = jnp.clip(fy1.astype(jnp.int32), 0, H - 1)
                r0 = cy0 * W
                r1 = cy1 * W
                idxv[0, sl] = r0 + cx0
                idxv[1, sl] = r0 + cx1
                idxv[2, sl] = r1 + cx0
                idxv[3, sl] = r1 + cx1
                zero = jnp.zeros((L,), jnp.float32)
                wv[0, sl] = jnp.where(mx0 & my0, wx0 * wy0, zero)
                wv[1, sl] = jnp.where(mx1 & my0, wx1 * wy0, zero)
                wv[2, sl] = jnp.where(mx0 & my1, wx0 * wy1, zero)
                wv[3, sl] = jnp.where(mx1 & my1, wx1 * wy1, zero)

            xb = x_hbm.at[b]
            cps = [
                pltpu.async_copy(
                    xb.at[idxv.at[k, pl.ds(0, size)]],
                    rowsv.at[pl.ds(k * CH, size)],
                    sem,
                )
                for k in range(4)
            ]
            for cp in cps:
                cp.wait()

            def group_body(g, carry2):
                gsl = pl.ds(g * L, L)
                w00 = wv[0, gsl]
                w01 = wv[1, gsl]
                w10 = wv[2, gsl]
                w11 = wv[3, gsl]
                for j in range(L):
                    lanes = jnp.full((L,), j, jnp.int32)
                    b00 = w00.at[lanes].get(mode="promise_in_bounds")
                    b01 = w01.at[lanes].get(mode="promise_in_bounds")
                    b10 = w10.at[lanes].get(mode="promise_in_bounds")
                    b11 = w11.at[lanes].get(mode="promise_in_bounds")
                    p = g * L + j
                    for cc in range(C // L):
                        csl = pl.ds(cc * L, L)
                        acc = rowsv[p, csl] * b00
                        acc += rowsv[CH + p, csl] * b01
                        acc += rowsv[2 * CH + p, csl] * b10
                        acc += rowsv[3 * CH + p, csl] * b11
                        outv[p, csl] = acc
                return carry2

            lax.fori_loop(0, size // L, group_body, 0)
            pltpu.sync_copy(
                outv.at[pl.ds(0, size)], out_hbm.at[b, pl.ds(lc * CH, size)]
            )

        def chunk_body(t, carry):
            c = wid * per_w + t
            b = jnp.int32(0)
            for bb in range(1, B):
                b = b + jnp.where(c >= bb * full_per_b, 1, 0)
            lc = c - b * full_per_b
            do_chunk(b, lc, CH)
            return carry

        lax.fori_loop(0, per_w, chunk_body, 0)

        if tail:
            @pl.when(wid < B)
            def _():
                do_chunk(wid, jnp.int32(full_per_b), tail)

    @jax.jit
    def run(x3, px, py):
        return interp(x3, px, py)

    return run


def kernel(x, pos, H, W):
    B, Hs, Ws, C = x.shape
    N = pos.shape[1]
    x3 = x.reshape(B, Hs * Ws, C)
    px = pos[..., 0].reshape(-1)
    py = pos[..., 1].reshape(-1)
    return _build(B, Hs, Ws, C, N)(x3, px, py)
